# baseline (speedup 1.0000x reference)
import math

import jax
import jax.numpy as jnp
from jax.experimental import pallas as pl
from jax.experimental.pallas import tpu as pltpu

D_MODEL = 1024
N_HEADS = 16
HEAD_DIM = D_MODEL // N_HEADS
B = 2
S = 4096
SAMPLED = max(1, min(S, int(5 * math.log(S + 1))))
BS = 512
OC = 512
NSEL = N_HEADS * SAMPLED
HP = 4
CW = HP * HEAD_DIM
RW = HP * SAMPLED
NQ = S // BS
NA = N_HEADS // HP
NO = S // OC
PH = NQ + 1 + NA + NO


def _body(x_ref, wq_ref, wk_ref, wv_ref, wo_ref, b_ref, bo_ref, out_ref,
          q_s, k_s, v_s, sp_s, vsum_s, idx_s, base_s, dout_s):
    s = pl.program_id(1)

    @pl.when(s < NQ)
    def _qkv():
        off = pl.multiple_of(s * BS, BS)
        x = x_ref[0].astype(jnp.bfloat16)
        tdims = (((1,), (1,)), ((), ()))
        q = jax.lax.dot_general(x, wq_ref[...], tdims,
                                preferred_element_type=jnp.float32)
        q = q + b_ref[0, 0]
        q_s[pl.ds(off, BS), :] = q.astype(jnp.bfloat16)
        k = jax.lax.dot_general(x, wk_ref[...], tdims,
                                preferred_element_type=jnp.float32)
        k_s[pl.ds(off, BS), :] = (k + b_ref[0, 1]).astype(jnp.bfloat16)
        v = jax.lax.dot_general(x, wv_ref[...], tdims,
                                preferred_element_type=jnp.float32)
        v = v + b_ref[0, 2]
        v_s[pl.ds(off, BS), :] = v.astype(jnp.bfloat16)
        q2 = (q * q).reshape(BS, N_HEADS, HEAD_DIM)
        sp_s[:, pl.ds(off, BS)] = jnp.sum(q2, axis=2).T

        @pl.when(s == 0)
        def _():
            vsum_s[...] = jnp.zeros((1, D_MODEL), jnp.float32)
        vsum_s[...] += jnp.sum(v, axis=0, keepdims=True)

    @pl.when(s == NQ)
    def _topk():
        vals = sp_s[...]
        col = jax.lax.broadcasted_iota(jnp.int32, (N_HEADS, S), 1)
        neg = jnp.float32(-jnp.inf)
        for t in range(SAMPLED):
            m = jnp.max(vals, axis=1, keepdims=True)
            arg = jnp.min(jnp.where(vals == m, col, S), axis=1,
                          keepdims=True)
            idx_s[:, t:t + 1] = arg
            vals = jnp.where(col == arg, neg, vals)
        mean_flat = vsum_s[...] * jnp.float32(1.0 / S)
        base = jax.lax.dot_general(mean_flat.astype(jnp.bfloat16),
                                   wo_ref[...], (((1,), (1,)), ((), ())),
                                   preferred_element_type=jnp.float32)
        base_s[...] = base + bo_ref[...]

    srow = jax.lax.broadcasted_iota(jnp.int32, (S, RW), 0)
    rblk = jax.lax.broadcasted_iota(jnp.int32, (RW, CW), 0) // SAMPLED
    cblk = jax.lax.broadcasted_iota(jnp.int32, (RW, CW), 1) // HEAD_DIM
    mask = rblk == cblk
    for g in range(NA):
        @pl.when(s == NQ + 1 + g)
        def _attn(g=g):
            lo = g * CW
            idx_cat = jnp.concatenate(
                [idx_s[g * HP + j:g * HP + j + 1, :] for j in range(HP)],
                axis=1)
            oht = (srow == idx_cat).astype(jnp.bfloat16)
            q4 = q_s[:, lo:lo + CW]
            selqf = jax.lax.dot_general(oht, q4, (((0,), (0,)), ((), ())),
                                        preferred_element_type=jnp.float32)
            selq = jnp.where(mask, selqf, 0.0).astype(jnp.bfloat16)
            scores = jax.lax.dot_general(selq, k_s[:, lo:lo + CW],
                                         (((1,), (1,)), ((), ())),
                                         preferred_element_type=jnp.float32)
            scores = scores * (HEAD_DIM ** (-0.5))
            m = jnp.max(scores, axis=1, keepdims=True)
            p = jnp.exp(scores - m)
            denom = jnp.sum(p, axis=1, keepdims=True)
            ctx = jax.lax.dot_general(p.astype(jnp.bfloat16),
                                      v_s[:, lo:lo + CW],
                                      (((1,), (0,)), ((), ())),
                                      preferred_element_type=jnp.float32)
            ctx = ctx / denom
            mean4 = vsum_s[0, lo:lo + CW] * jnp.float32(1.0 / S)
            delta = jnp.where(mask, ctx - mean4, 0.0).astype(jnp.bfloat16)
            dout = jax.lax.dot_general(delta, wo_ref[:, lo:lo + CW],
                                       (((1,), (1,)), ((), ())),
                                       preferred_element_type=jnp.float32)
            dout_s[pl.ds(g * RW, RW), :] = dout.astype(jnp.bfloat16)

    @pl.when(s >= NQ + 1 + NA)
    def _scatter():
        c = s - (NQ + 1 + NA)
        idxf = jnp.concatenate(
            [idx_s[h:h + 1, :] for h in range(N_HEADS)], axis=1)
        row = jax.lax.broadcasted_iota(jnp.int32, (OC, NSEL), 0) + c * OC
        oh = (row == idxf).astype(jnp.bfloat16)
        scat = jax.lax.dot_general(oh, dout_s[...], (((1,), (0,)), ((), ())),
                                   preferred_element_type=jnp.float32)
        out_ref[0] = scat + base_s[...]


def _clip(v, hi):
    return jnp.minimum(v, hi)


@jax.jit
def kernel(x, Wq, bq, Wk, bk, Wv, bv, Wo, bo):
    wq_bf = Wq.astype(jnp.bfloat16)
    wk_bf = Wk.astype(jnp.bfloat16)
    wv_bf = Wv.astype(jnp.bfloat16)
    wo_bf = Wo.astype(jnp.bfloat16)
    b3 = jnp.stack([bq, bk, bv]).reshape(1, 3, D_MODEL)
    bo2 = bo.reshape(1, D_MODEL)

    return pl.pallas_call(
        _body,
        grid=(B, PH),
        in_specs=[
            pl.BlockSpec((1, BS, D_MODEL),
                         lambda b, s: (b, _clip(s, NQ - 1), 0)),
            pl.BlockSpec((D_MODEL, D_MODEL), lambda b, s: (0, 0)),
            pl.BlockSpec((D_MODEL, D_MODEL), lambda b, s: (0, 0)),
            pl.BlockSpec((D_MODEL, D_MODEL), lambda b, s: (0, 0)),
            pl.BlockSpec((D_MODEL, D_MODEL), lambda b, s: (0, 0)),
            pl.BlockSpec((1, 3, D_MODEL), lambda b, s: (0, 0, 0)),
            pl.BlockSpec((1, D_MODEL), lambda b, s: (0, 0)),
        ],
        out_specs=pl.BlockSpec(
            (1, OC, D_MODEL),
            lambda b, s: (b, _clip(jnp.maximum(s - (NQ + 1 + NA), 0),
                                   NO - 1), 0)),
        out_shape=jax.ShapeDtypeStruct((B, S, D_MODEL), jnp.float32),
        scratch_shapes=[
            pltpu.VMEM((S, D_MODEL), jnp.bfloat16),
            pltpu.VMEM((S, D_MODEL), jnp.bfloat16),
            pltpu.VMEM((S, D_MODEL), jnp.bfloat16),
            pltpu.VMEM((N_HEADS, S), jnp.float32),
            pltpu.VMEM((1, D_MODEL), jnp.float32),
            pltpu.VMEM((N_HEADS, SAMPLED), jnp.int32),
            pltpu.VMEM((1, D_MODEL), jnp.float32),
            pltpu.VMEM((NSEL, D_MODEL), jnp.bfloat16),
        ],
    )(x, wq_bf, wk_bf, wv_bf, wo_bf, b3, bo2)


# speedup vs baseline: 1.0322x; 1.0322x over previous
import math

import jax
import jax.numpy as jnp
from jax.experimental import pallas as pl
from jax.experimental.pallas import tpu as pltpu

D_MODEL = 1024
N_HEADS = 16
HEAD_DIM = D_MODEL // N_HEADS
B = 2
S = 4096
SAMPLED = max(1, min(S, int(5 * math.log(S + 1))))
BS = 1024
OC = 512
NSEL = N_HEADS * SAMPLED


def _qkv_body(x_ref, wq_ref, wk_ref, wv_ref, b_ref,
              q_ref, k_ref, v_ref, sp_ref, vsum_ref):
    s = pl.program_id(1)
    x = x_ref[0].astype(jnp.bfloat16)
    tdims = (((1,), (1,)), ((), ()))
    q = jax.lax.dot_general(x, wq_ref[...], tdims,
                            preferred_element_type=jnp.float32)
    q = q + b_ref[0, 0]
    q_ref[0] = q.astype(jnp.bfloat16)
    k = jax.lax.dot_general(x, wk_ref[...], tdims,
                            preferred_element_type=jnp.float32)
    k_ref[0] = (k + b_ref[0, 1]).astype(jnp.bfloat16)
    v = jax.lax.dot_general(x, wv_ref[...], tdims,
                            preferred_element_type=jnp.float32)
    v = v + b_ref[0, 2]
    v_ref[0] = v.astype(jnp.bfloat16)
    q2 = (q * q).reshape(BS, N_HEADS, HEAD_DIM)
    sp = jnp.sum(q2, axis=2)
    sp_ref[0] = sp.T

    @pl.when(s == 0)
    def _():
        vsum_ref[0] = jnp.zeros((1, D_MODEL), jnp.float32)
    vsum_ref[0] += jnp.sum(v, axis=0, keepdims=True)


def _qkv_stage(x, wq, wk, wv, b3):
    grid = (B, S // BS)
    return pl.pallas_call(
        _qkv_body,
        grid=grid,
        in_specs=[
            pl.BlockSpec((1, BS, D_MODEL), lambda b, s: (b, s, 0)),
            pl.BlockSpec((D_MODEL, D_MODEL), lambda b, s: (0, 0)),
            pl.BlockSpec((D_MODEL, D_MODEL), lambda b, s: (0, 0)),
            pl.BlockSpec((D_MODEL, D_MODEL), lambda b, s: (0, 0)),
            pl.BlockSpec((1, 3, D_MODEL), lambda b, s: (0, 0, 0)),
        ],
        out_specs=[
            pl.BlockSpec((1, BS, D_MODEL), lambda b, s: (b, s, 0)),
            pl.BlockSpec((1, BS, D_MODEL), lambda b, s: (b, s, 0)),
            pl.BlockSpec((1, BS, D_MODEL), lambda b, s: (b, s, 0)),
            pl.BlockSpec((1, N_HEADS, BS), lambda b, s: (b, 0, s)),
            pl.BlockSpec((1, 1, D_MODEL), lambda b, s: (b, 0, 0)),
        ],
        out_shape=[
            jax.ShapeDtypeStruct((B, S, D_MODEL), jnp.bfloat16),
            jax.ShapeDtypeStruct((B, S, D_MODEL), jnp.bfloat16),
            jax.ShapeDtypeStruct((B, S, D_MODEL), jnp.bfloat16),
            jax.ShapeDtypeStruct((B, N_HEADS, S), jnp.float32),
            jax.ShapeDtypeStruct((B, 1, D_MODEL), jnp.float32),
        ],
    )(x, wq, wk, wv, b3)


def _topk_body(sp_ref, idx_ref):
    vals = sp_ref[...]
    col = jax.lax.broadcasted_iota(jnp.int32, vals.shape, 1)
    neg = jnp.float32(-jnp.inf)
    for t in range(SAMPLED):
        m = jnp.max(vals, axis=1, keepdims=True)
        eq = vals == m
        arg = jnp.min(jnp.where(eq, col, S), axis=1, keepdims=True)
        idx_ref[:, t:t + 1] = arg
        vals = jnp.where(col == arg, neg, vals)


def _topk_stage(sp):
    return pl.pallas_call(
        _topk_body,
        out_shape=jax.ShapeDtypeStruct((B * N_HEADS, SAMPLED), jnp.int32),
    )(sp)


HP = 8
CW = HP * HEAD_DIM
RW = HP * SAMPLED


def _attn_body(idx_ref, q_ref, k_ref, v_ref, wo_ref, vsum_ref, dout_ref):
    srow = jax.lax.broadcasted_iota(jnp.int32, (S, RW), 0)
    idx_cat = jnp.concatenate([idx_ref[j] for j in range(HP)], axis=1)
    oht = (srow == idx_cat).astype(jnp.bfloat16)
    rblk = jax.lax.broadcasted_iota(jnp.int32, (RW, CW), 0) // SAMPLED
    cblk = jax.lax.broadcasted_iota(jnp.int32, (RW, CW), 1) // HEAD_DIM
    mask = rblk == cblk
    selqf = jax.lax.dot_general(oht, q_ref[0], (((0,), (0,)), ((), ())),
                                preferred_element_type=jnp.float32)
    selq = jnp.where(mask, selqf, 0.0).astype(jnp.bfloat16)
    scores = jax.lax.dot_general(selq, k_ref[0], (((1,), (1,)), ((), ())),
                                 preferred_element_type=jnp.float32)
    scores = scores * (HEAD_DIM ** (-0.5))
    m = jnp.max(scores, axis=1, keepdims=True)
    p = jnp.exp(scores - m)
    denom = jnp.sum(p, axis=1, keepdims=True)
    ctx = jax.lax.dot_general(p.astype(jnp.bfloat16), v_ref[0],
                              (((1,), (0,)), ((), ())),
                              preferred_element_type=jnp.float32)
    ctx = ctx / denom
    mean4 = vsum_ref[0, 0] * jnp.float32(1.0 / S)
    delta = jnp.where(mask, ctx - mean4, 0.0).astype(jnp.bfloat16)
    dout = jax.lax.dot_general(delta, wo_ref[...], (((1,), (1,)), ((), ())),
                               preferred_element_type=jnp.float32)
    dout = dout.astype(jnp.bfloat16)
    for j in range(HP):
        dout_ref[0, j] = dout[j * SAMPLED:(j + 1) * SAMPLED]


def _attn_stage(idx3, q, k, v, wot, vsum):
    grid = (B, N_HEADS // HP)
    return pl.pallas_call(
        _attn_body,
        grid=grid,
        in_specs=[
            pl.BlockSpec((HP, 1, SAMPLED),
                         lambda b, g: (b * (N_HEADS // HP) + g, 0, 0)),
            pl.BlockSpec((1, S, CW), lambda b, g: (b, 0, g)),
            pl.BlockSpec((1, S, CW), lambda b, g: (b, 0, g)),
            pl.BlockSpec((1, S, CW), lambda b, g: (b, 0, g)),
            pl.BlockSpec((D_MODEL, CW), lambda b, g: (0, g)),
            pl.BlockSpec((1, 1, CW), lambda b, g: (b, 0, g)),
        ],
        out_specs=pl.BlockSpec((1, HP, SAMPLED, D_MODEL),
                               lambda b, g: (b, g, 0, 0)),
        out_shape=jax.ShapeDtypeStruct((B, N_HEADS, SAMPLED, D_MODEL),
                                       jnp.bfloat16),
    )(idx3, q, k, v, wot, vsum)


def _assemble_body(idxf_ref, dout_ref, vsum_ref, wo_ref, bo_ref, out_ref,
                   base_ref):
    c = pl.program_id(1)

    @pl.when(c == 0)
    def _():
        mean_flat = vsum_ref[0] * jnp.float32(1.0 / S)
        base = jax.lax.dot_general(mean_flat.astype(jnp.bfloat16),
                                   wo_ref[...], (((1,), (1,)), ((), ())),
                                   preferred_element_type=jnp.float32)
        base_ref[...] = base + bo_ref[...]

    row = jax.lax.broadcasted_iota(jnp.int32, (OC, NSEL), 0) + c * OC
    oh = (row == idxf_ref[0]).astype(jnp.bfloat16)
    scat = jax.lax.dot_general(oh, dout_ref[0], (((1,), (0,)), ((), ())),
                               preferred_element_type=jnp.float32)
    out_ref[0] = scat + base_ref[...]


def _assemble_stage(idxf, dout_flat, vsum, wot, bo2):
    grid = (B, S // OC)
    return pl.pallas_call(
        _assemble_body,
        grid=grid,
        in_specs=[
            pl.BlockSpec((1, 1, NSEL), lambda b, c: (b, 0, 0)),
            pl.BlockSpec((1, NSEL, D_MODEL), lambda b, c: (b, 0, 0)),
            pl.BlockSpec((1, 1, D_MODEL), lambda b, c: (b, 0, 0)),
            pl.BlockSpec((D_MODEL, D_MODEL), lambda b, c: (0, 0)),
            pl.BlockSpec((1, D_MODEL), lambda b, c: (0, 0)),
        ],
        out_specs=pl.BlockSpec((1, OC, D_MODEL), lambda b, c: (b, c, 0)),
        out_shape=jax.ShapeDtypeStruct((B, S, D_MODEL), jnp.float32),
        scratch_shapes=[pltpu.VMEM((1, D_MODEL), jnp.float32)],
    )(idxf, dout_flat, vsum, wot, bo2)


@jax.jit
def kernel(x, Wq, bq, Wk, bk, Wv, bv, Wo, bo):
    wq_bf = Wq.astype(jnp.bfloat16)
    wk_bf = Wk.astype(jnp.bfloat16)
    wv_bf = Wv.astype(jnp.bfloat16)
    wo_bf = Wo.astype(jnp.bfloat16)
    b3 = jnp.stack([bq, bk, bv]).reshape(1, 3, D_MODEL)
    bo2 = bo.reshape(1, D_MODEL)

    q, k, v, sp, vsum = _qkv_stage(x, wq_bf, wk_bf, wv_bf, b3)
    top_idx = _topk_stage(sp.reshape(B * N_HEADS, S))
    idx3 = top_idx.reshape(B * N_HEADS, 1, SAMPLED)
    dout = _attn_stage(idx3, q, k, v, wo_bf, vsum)
    idxf = top_idx.reshape(B, 1, NSEL)
    dout_flat = dout.reshape(B, NSEL, D_MODEL)
    return _assemble_stage(idxf, dout_flat, vsum, wo_bf, bo2)


# speedup vs baseline: 1.0528x; 1.0200x over previous
import math

import jax
import jax.numpy as jnp
from jax.experimental import pallas as pl
from jax.experimental.pallas import tpu as pltpu

D_MODEL = 1024
N_HEADS = 16
HEAD_DIM = D_MODEL // N_HEADS
B = 2
S = 4096
SAMPLED = max(1, min(S, int(5 * math.log(S + 1))))
BS = 1024
OC = 1024
NSEL = N_HEADS * SAMPLED


def _qkv_body(x_ref, wq_ref, wk_ref, wv_ref, b_ref,
              q_ref, k_ref, v_ref, sp_ref, vsum_ref):
    s = pl.program_id(1)
    x = x_ref[0].astype(jnp.bfloat16)
    tdims = (((1,), (1,)), ((), ()))
    q = jax.lax.dot_general(x, wq_ref[...], tdims,
                            preferred_element_type=jnp.float32)
    q = q + b_ref[0, 0]
    q_ref[0] = q.astype(jnp.bfloat16)
    k = jax.lax.dot_general(x, wk_ref[...], tdims,
                            preferred_element_type=jnp.float32)
    k_ref[0] = (k + b_ref[0, 1]).astype(jnp.bfloat16)
    v = jax.lax.dot_general(x, wv_ref[...], tdims,
                            preferred_element_type=jnp.float32)
    v = v + b_ref[0, 2]
    v_ref[0] = v.astype(jnp.bfloat16)
    q2 = (q * q).reshape(BS, N_HEADS, HEAD_DIM)
    sp = jnp.sum(q2, axis=2)
    sp_ref[0] = sp.T

    @pl.when(s == 0)
    def _():
        vsum_ref[0] = jnp.zeros((1, D_MODEL), jnp.float32)
    vsum_ref[0] += jnp.sum(v, axis=0, keepdims=True)


def _qkv_stage(x, wq, wk, wv, b3):
    grid = (B, S // BS)
    return pl.pallas_call(
        _qkv_body,
        grid=grid,
        in_specs=[
            pl.BlockSpec((1, BS, D_MODEL), lambda b, s: (b, s, 0)),
            pl.BlockSpec((D_MODEL, D_MODEL), lambda b, s: (0, 0)),
            pl.BlockSpec((D_MODEL, D_MODEL), lambda b, s: (0, 0)),
            pl.BlockSpec((D_MODEL, D_MODEL), lambda b, s: (0, 0)),
            pl.BlockSpec((1, 3, D_MODEL), lambda b, s: (0, 0, 0)),
        ],
        out_specs=[
            pl.BlockSpec((1, BS, D_MODEL), lambda b, s: (b, s, 0)),
            pl.BlockSpec((1, BS, D_MODEL), lambda b, s: (b, s, 0)),
            pl.BlockSpec((1, BS, D_MODEL), lambda b, s: (b, s, 0)),
            pl.BlockSpec((1, N_HEADS, BS), lambda b, s: (b, 0, s)),
            pl.BlockSpec((1, 1, D_MODEL), lambda b, s: (b, 0, 0)),
        ],
        out_shape=[
            jax.ShapeDtypeStruct((B, S, D_MODEL), jnp.bfloat16),
            jax.ShapeDtypeStruct((B, S, D_MODEL), jnp.bfloat16),
            jax.ShapeDtypeStruct((B, S, D_MODEL), jnp.bfloat16),
            jax.ShapeDtypeStruct((B, N_HEADS, S), jnp.float32),
            jax.ShapeDtypeStruct((B, 1, D_MODEL), jnp.float32),
        ],
    )(x, wq, wk, wv, b3)


def _topk_body(sp_ref, idx_ref):
    vals = sp_ref[...]
    col = jax.lax.broadcasted_iota(jnp.int32, vals.shape, 1)
    neg = jnp.float32(-jnp.inf)
    for t in range(SAMPLED):
        m = jnp.max(vals, axis=1, keepdims=True)
        eq = vals == m
        arg = jnp.min(jnp.where(eq, col, S), axis=1, keepdims=True)
        idx_ref[:, t:t + 1] = arg
        vals = jnp.where(col == arg, neg, vals)


def _topk_stage(sp):
    return pl.pallas_call(
        _topk_body,
        out_shape=jax.ShapeDtypeStruct((B * N_HEADS, SAMPLED), jnp.int32),
    )(sp)


HP = 8
CW = HP * HEAD_DIM
RW = HP * SAMPLED


def _attn_body(idx_ref, q_ref, k_ref, v_ref, wo_ref, vsum_ref, dout_ref):
    srow = jax.lax.broadcasted_iota(jnp.int32, (S, RW), 0)
    idx_cat = jnp.concatenate([idx_ref[j] for j in range(HP)], axis=1)
    oht = (srow == idx_cat).astype(jnp.bfloat16)
    rblk = jax.lax.broadcasted_iota(jnp.int32, (RW, CW), 0) // SAMPLED
    cblk = jax.lax.broadcasted_iota(jnp.int32, (RW, CW), 1) // HEAD_DIM
    mask = rblk == cblk
    selqf = jax.lax.dot_general(oht, q_ref[0], (((0,), (0,)), ((), ())),
                                preferred_element_type=jnp.float32)
    selq = jnp.where(mask, selqf, 0.0).astype(jnp.bfloat16)
    scores = jax.lax.dot_general(selq, k_ref[0], (((1,), (1,)), ((), ())),
                                 preferred_element_type=jnp.float32)
    scores = scores * (HEAD_DIM ** (-0.5))
    m = jnp.max(scores, axis=1, keepdims=True)
    p = jnp.exp(scores - m)
    denom = jnp.sum(p, axis=1, keepdims=True)
    ctx = jax.lax.dot_general(p.astype(jnp.bfloat16), v_ref[0],
                              (((1,), (0,)), ((), ())),
                              preferred_element_type=jnp.float32)
    ctx = ctx / denom
    mean4 = vsum_ref[0, 0] * jnp.float32(1.0 / S)
    delta = jnp.where(mask, ctx - mean4, 0.0).astype(jnp.bfloat16)
    dout = jax.lax.dot_general(delta, wo_ref[...], (((1,), (1,)), ((), ())),
                               preferred_element_type=jnp.float32)
    dout = dout.astype(jnp.bfloat16)
    for j in range(HP):
        dout_ref[0, j] = dout[j * SAMPLED:(j + 1) * SAMPLED]


def _attn_stage(idx3, q, k, v, wot, vsum):
    grid = (B, N_HEADS // HP)
    return pl.pallas_call(
        _attn_body,
        grid=grid,
        in_specs=[
            pl.BlockSpec((HP, 1, SAMPLED),
                         lambda b, g: (b * (N_HEADS // HP) + g, 0, 0)),
            pl.BlockSpec((1, S, CW), lambda b, g: (b, 0, g)),
            pl.BlockSpec((1, S, CW), lambda b, g: (b, 0, g)),
            pl.BlockSpec((1, S, CW), lambda b, g: (b, 0, g)),
            pl.BlockSpec((D_MODEL, CW), lambda b, g: (0, g)),
            pl.BlockSpec((1, 1, CW), lambda b, g: (b, 0, g)),
        ],
        out_specs=pl.BlockSpec((1, HP, SAMPLED, D_MODEL),
                               lambda b, g: (b, g, 0, 0)),
        out_shape=jax.ShapeDtypeStruct((B, N_HEADS, SAMPLED, D_MODEL),
                                       jnp.bfloat16),
    )(idx3, q, k, v, wot, vsum)


def _assemble_body(idxf_ref, dout_ref, vsum_ref, wo_ref, bo_ref, out_ref,
                   base_ref):
    c = pl.program_id(1)

    @pl.when(c == 0)
    def _():
        mean_flat = vsum_ref[0] * jnp.float32(1.0 / S)
        base = jax.lax.dot_general(mean_flat.astype(jnp.bfloat16),
                                   wo_ref[...], (((1,), (1,)), ((), ())),
                                   preferred_element_type=jnp.float32)
        base_ref[...] = base + bo_ref[...]

    row = jax.lax.broadcasted_iota(jnp.int32, (OC, NSEL), 0) + c * OC
    oh = (row == idxf_ref[0]).astype(jnp.bfloat16)
    scat = jax.lax.dot_general(oh, dout_ref[0], (((1,), (0,)), ((), ())),
                               preferred_element_type=jnp.float32)
    out_ref[0] = scat + base_ref[...]


def _assemble_stage(idxf, dout_flat, vsum, wot, bo2):
    grid = (B, S // OC)
    return pl.pallas_call(
        _assemble_body,
        grid=grid,
        in_specs=[
            pl.BlockSpec((1, 1, NSEL), lambda b, c: (b, 0, 0)),
            pl.BlockSpec((1, NSEL, D_MODEL), lambda b, c: (b, 0, 0)),
            pl.BlockSpec((1, 1, D_MODEL), lambda b, c: (b, 0, 0)),
            pl.BlockSpec((D_MODEL, D_MODEL), lambda b, c: (0, 0)),
            pl.BlockSpec((1, D_MODEL), lambda b, c: (0, 0)),
        ],
        out_specs=pl.BlockSpec((1, OC, D_MODEL), lambda b, c: (b, c, 0)),
        out_shape=jax.ShapeDtypeStruct((B, S, D_MODEL), jnp.float32),
        scratch_shapes=[pltpu.VMEM((1, D_MODEL), jnp.float32)],
    )(idxf, dout_flat, vsum, wot, bo2)


@jax.jit
def kernel(x, Wq, bq, Wk, bk, Wv, bv, Wo, bo):
    wq_bf = Wq.astype(jnp.bfloat16)
    wk_bf = Wk.astype(jnp.bfloat16)
    wv_bf = Wv.astype(jnp.bfloat16)
    wo_bf = Wo.astype(jnp.bfloat16)
    b3 = jnp.stack([bq, bk, bv]).reshape(1, 3, D_MODEL)
    bo2 = bo.reshape(1, D_MODEL)

    q, k, v, sp, vsum = _qkv_stage(x, wq_bf, wk_bf, wv_bf, b3)
    top_idx = _topk_stage(sp.reshape(B * N_HEADS, S))
    idx3 = top_idx.reshape(B * N_HEADS, 1, SAMPLED)
    dout = _attn_stage(idx3, q, k, v, wo_bf, vsum)
    idxf = top_idx.reshape(B, 1, NSEL)
    dout_flat = dout.reshape(B, NSEL, D_MODEL)
    return _assemble_stage(idxf, dout_flat, vsum, wo_bf, bo2)
